# Initial kernel scaffold; baseline (speedup 1.0000x reference)
import jax
import jax.numpy as jnp
from jax.experimental import pallas as pl


def kernel(compressed_matrix):
    raise NotImplementedError("write your pallas kernel here")


# speedup vs baseline: 126.6224x; 126.6224x over previous
import functools
import math

import jax
import jax.numpy as jnp
from jax import lax
from jax.experimental import pallas as pl
from jax.experimental.pallas import tpu as pltpu
from jax.experimental.pallas import tpu_sc as plsc

N = 4096
M = N * (N - 1) // 2
B = 128
NBLK = N // B
NC, NS, L = 2, 16, 16
NW = NC * NS
BLOCKS_PER_W = NBLK * NBLK // NW
G = 128
F = G
TAIL = 2 * G
MP = F + M + TAIL
NSEG = 2
QC = B // L


def _body(comp_ref, out_ref, idx_ref, stage_ref, outbuf_ref, shift_ref, sem):
    wid = lax.axis_index("s") * NC + lax.axis_index("c")
    iota = lax.iota(jnp.int32, L)

    def seg_start(g, mx):
        off = g * N - lax.shift_right_logical(g * (g + 1), 1)
        return off + (F + mx - 1) - g

    def run_block(k, carry):
        b = k * NW + wid
        bi = lax.shift_right_logical(b, 5)
        bj = lax.bitwise_and(b, NBLK - 1)
        i0 = bi * B
        j0 = bj * B
        mn = jnp.minimum(i0, j0)
        mx = jnp.maximum(i0, j0)

        def build(t16, c_):
            tv = t16 * L + iota
            s = seg_start(mn + tv, mx)
            a = lax.shift_right_logical(s, 7)
            shift_ref[pl.ds(t16 * L, L)] = lax.bitwise_and(s, G - 1)
            for c in range(NSEG):
                idx_ref[c, pl.ds(t16 * L, L)] = a + c
            return c_

        lax.fori_loop(0, QC, build, 0)

        copies = [
            pltpu.async_copy(comp_ref.at[idx_ref.at[c]], stage_ref.at[c], sem)
            for c in range(NSEG)
        ]
        for cp in copies:
            cp.wait()


        def gather_stage(cvec, tvec, lvec):
            return plsc.load_gather(stage_ref, [cvec, tvec, lvec])

        def upper_row(r, c_):
            sh = lax.bitwise_and(seg_start(mn + r, mx), G - 1)
            rv = jnp.full((L,), r, dtype=jnp.int32)

            def chunk(qc, c2_):
                x = sh + qc * L + iota
                v = gather_stage(
                    lax.shift_right_logical(x, 7), rv, lax.bitwise_and(x, G - 1)
                )
                outbuf_ref[r, pl.ds(qc * L, L)] = v
                return c2_

            return lax.fori_loop(0, QC, chunk, c_)

        def lower_chunk(qc, c_):
            qv = qc * L + iota
            shq = shift_ref[pl.ds(qc * L, L)]

            def row(r, c2_):
                x = shq + r
                v = gather_stage(
                    lax.shift_right_logical(x, 7), qv, lax.bitwise_and(x, G - 1)
                )
                outbuf_ref[r, pl.ds(qc * L, L)] = v
                return c2_

            return lax.fori_loop(0, B, row, c_)

        def diag_chunk(qc, c_):
            qv = qc * L + iota
            shq = shift_ref[pl.ds(qc * L, L)]

            def row(r, c2_):
                sh = lax.bitwise_and(seg_start(mn + r, mx), G - 1)
                rv = jnp.full((L,), r, dtype=jnp.int32)
                xu = sh + qv
                vu = gather_stage(
                    lax.shift_right_logical(xu, 7), rv, lax.bitwise_and(xu, G - 1)
                )
                xl = shq + r
                vl = gather_stage(
                    lax.shift_right_logical(xl, 7), qv, lax.bitwise_and(xl, G - 1)
                )
                val = jnp.where(qv > r, vu, jnp.where(qv < r, vl, 1.0))
                outbuf_ref[r, pl.ds(qc * L, L)] = val
                return c2_

            return lax.fori_loop(0, B, row, c_)

        @pl.when(bi < bj)
        def _():
            lax.fori_loop(0, B, upper_row, 0)

        @pl.when(bi > bj)
        def _():
            lax.fori_loop(0, QC, lower_chunk, 0)

        @pl.when(bi == bj)
        def _():
            lax.fori_loop(0, QC, diag_chunk, 0)

        pltpu.sync_copy(outbuf_ref, out_ref.at[pl.ds(i0, B), pl.ds(j0, B)])
        return carry

    lax.fori_loop(0, BLOCKS_PER_W, run_block, 0)


@jax.jit
def kernel(compressed_matrix):
    comp_pad = jnp.concatenate(
        [
            jnp.zeros((F,), jnp.float32),
            compressed_matrix,
            jnp.zeros((TAIL,), jnp.float32),
        ]
    ).reshape(MP // G, G)

    mesh = plsc.VectorSubcoreMesh(core_axis_name="c", subcore_axis_name="s")
    run = pl.kernel(
        _body,
        out_type=jax.ShapeDtypeStruct((N, N), jnp.float32),
        mesh=mesh,
        scratch_types=[
            pltpu.VMEM((NSEG, B), jnp.int32),
            pltpu.VMEM((NSEG, B, G), jnp.float32),
            pltpu.VMEM((B, B), jnp.float32),
            pltpu.VMEM((B,), jnp.int32),
            pltpu.SemaphoreType.DMA,
        ],
        compiler_params=pltpu.CompilerParams(needs_layout_passes=False),
    )
    return run(comp_pad)


# speedup vs baseline: 144.7905x; 1.1435x over previous
import functools
import math

import jax
import jax.numpy as jnp
from jax import lax
from jax.experimental import pallas as pl
from jax.experimental.pallas import tpu as pltpu
from jax.experimental.pallas import tpu_sc as plsc

N = 4096
M = N * (N - 1) // 2
B = 128
NBLK = N // B
NC, NS, L = 2, 16, 16
NW = NC * NS
BLOCKS_PER_W = NBLK * NBLK // NW
G = 128
F = G
TAIL = 2 * G
MP = F + M + TAIL
NSEG = 2
QC = B // L


def _body(comp_ref, out_ref, idx_ref, stage_ref, outbuf_ref, shift_ref, sem):
    wid = lax.axis_index("s") * NC + lax.axis_index("c")
    iota = lax.iota(jnp.int32, L)
    qvs = [c * L + iota for c in range(QC)]

    def seg_start(g, mx):
        off = g * N - lax.shift_right_logical(g * (g + 1), 1)
        return off + (F + mx - 1) - g

    def run_block(k, carry):
        b = k * NW + wid
        bi = lax.shift_right_logical(b, 5)
        bj = lax.bitwise_and(b, NBLK - 1)
        i0 = bi * B
        j0 = bj * B
        mn = jnp.minimum(i0, j0)
        mx = jnp.maximum(i0, j0)

        def build(t16, c_):
            s = seg_start(mn + t16 * L + iota, mx)
            a = lax.shift_right_logical(s, 7)
            shift_ref[pl.ds(t16 * L, L)] = lax.bitwise_and(s, G - 1)
            for c in range(NSEG):
                idx_ref[c, pl.ds(t16 * L, L)] = a + c
            return c_

        lax.fori_loop(0, QC, build, 0)

        copies = [
            pltpu.async_copy(
                comp_ref.at[idx_ref.at[c]],
                stage_ref.at[:, pl.ds(c * G, G)],
                sem,
            )
            for c in range(NSEG)
        ]
        for cp in copies:
            cp.wait()

        sh_v = [shift_ref[pl.ds(c * L, L)] for c in range(QC)]

        def upper_row(r, c_):
            sh = lax.bitwise_and(seg_start(mn + r, mx), G - 1)
            rv = jnp.full((L,), r, dtype=jnp.int32)
            for c in range(QC):
                v = plsc.load_gather(stage_ref, [rv, sh + qvs[c]])
                outbuf_ref[r, pl.ds(c * L, L)] = v
            return c_

        def lower_row(r, c_):
            for c in range(QC):
                v = plsc.load_gather(stage_ref, [qvs[c], sh_v[c] + r])
                outbuf_ref[r, pl.ds(c * L, L)] = v
            return c_

        def diag_row(r, c_):
            sh = lax.bitwise_and(seg_start(mn + r, mx), G - 1)
            rv = jnp.full((L,), r, dtype=jnp.int32)
            for c in range(QC):
                vu = plsc.load_gather(stage_ref, [rv, sh + qvs[c]])
                vl = plsc.load_gather(stage_ref, [qvs[c], sh_v[c] + r])
                val = jnp.where(qvs[c] > r, vu, jnp.where(qvs[c] < r, vl, 1.0))
                outbuf_ref[r, pl.ds(c * L, L)] = val
            return c_

        @pl.when(bi < bj)
        def _():
            lax.fori_loop(0, B, upper_row, 0)

        @pl.when(bi > bj)
        def _():
            lax.fori_loop(0, B, lower_row, 0)

        @pl.when(bi == bj)
        def _():
            lax.fori_loop(0, B, diag_row, 0)

        pltpu.sync_copy(outbuf_ref, out_ref.at[pl.ds(i0, B), pl.ds(j0, B)])
        return carry

    lax.fori_loop(0, BLOCKS_PER_W, run_block, 0)


@jax.jit
def kernel(compressed_matrix):
    comp_pad = jnp.concatenate(
        [
            jnp.zeros((F,), jnp.float32),
            compressed_matrix,
            jnp.zeros((TAIL,), jnp.float32),
        ]
    ).reshape(MP // G, G)

    mesh = plsc.VectorSubcoreMesh(core_axis_name="c", subcore_axis_name="s")
    run = pl.kernel(
        _body,
        out_type=jax.ShapeDtypeStruct((N, N), jnp.float32),
        mesh=mesh,
        scratch_types=[
            pltpu.VMEM((NSEG, B), jnp.int32),
            pltpu.VMEM((B, NSEG * G), jnp.float32),
            pltpu.VMEM((B, B), jnp.float32),
            pltpu.VMEM((B,), jnp.int32),
            pltpu.SemaphoreType.DMA,
        ],
        compiler_params=pltpu.CompilerParams(needs_layout_passes=False),
    )
    return run(comp_pad)


# speedup vs baseline: 230.4352x; 1.5915x over previous
import functools
import math

import jax
import jax.numpy as jnp
from jax import lax
from jax.experimental import pallas as pl
from jax.experimental.pallas import tpu as pltpu
from jax.experimental.pallas import tpu_sc as plsc

N = 4096
M = N * (N - 1) // 2
B = 128
NBLK = N // B
NC, NS, L = 2, 16, 16
NW = NC * NS
BLOCKS_PER_W = NBLK * NBLK // NW
G = 128
ROWS = M // G
NSEG = 2
QC = B // L
SEG_BYTES = B * G * 4


def _body(comp_ref, out_ref, idx_ref, stage_ref, outbuf_ref, shift_ref,
          ssem0, ssem1, osem0, osem1):
    wid = lax.axis_index("s") * NC + lax.axis_index("c")
    iota = lax.iota(jnp.int32, L)
    qvs = [c * L + iota for c in range(QC)]
    ssems = (ssem0, ssem1)
    osems = (osem0, osem1)

    def seg_start(g, mx):
        off = g * N - lax.shift_right_logical(g * (g + 1), 1)
        return off + (mx - 1) - g

    def block_coords(blk):
        b = blk * NW + wid
        bi = lax.shift_right_logical(b, 5)
        bj = lax.bitwise_and(b, NBLK - 1)
        return bi * B, bj * B

    def build_and_fire(blk, p):
        i0, j0 = block_coords(blk)
        mn = jnp.minimum(i0, j0)
        mx = jnp.maximum(i0, j0)

        def build(t16, c_):
            s = seg_start(mn + t16 * L + iota, mx)
            a = jnp.maximum(lax.shift_right_arithmetic(s, 7), 0)
            shift_ref[p, pl.ds(t16 * L, L)] = s - lax.shift_left(a, 7)
            idx_ref[p, 0, pl.ds(t16 * L, L)] = a
            idx_ref[p, 1, pl.ds(t16 * L, L)] = jnp.minimum(a + 1, ROWS - 1)
            return c_

        lax.fori_loop(0, QC, build, 0)
        for c in range(NSEG):
            pltpu.make_async_copy(
                comp_ref.at[idx_ref.at[p, c]],
                stage_ref.at[pl.ds(p * B, B), pl.ds(c * G, G)],
                ssems[p],
            ).start()

    def wait_stage(p):
        for c in range(NSEG):
            pltpu.make_async_copy(
                comp_ref.at[idx_ref.at[p, c]],
                stage_ref.at[pl.ds(p * B, B), pl.ds(c * G, G)],
                ssems[p],
            ).wait()

    def out_copy(blk, p):
        i0, j0 = block_coords(blk)
        return pltpu.make_async_copy(
            outbuf_ref.at[pl.ds(p * B, B)],
            out_ref.at[pl.ds(i0, B), pl.ds(j0, B)],
            osems[p],
        )

    def compute_block(blk, p):
        i0, j0 = block_coords(blk)
        bi_lt = i0 < j0
        bi_gt = i0 > j0
        mn = jnp.minimum(i0, j0)
        mx = jnp.maximum(i0, j0)
        pB = p * B
        sh_v = [shift_ref[p, pl.ds(c * L, L)] for c in range(QC)]

        def upper_row(r, c_):
            sh = lax.bitwise_and(seg_start(mn + r, mx), G - 1)
            rv = jnp.full((L,), pB + r, dtype=jnp.int32)
            for c in range(QC):
                v = plsc.load_gather(stage_ref, [rv, sh + qvs[c]])
                outbuf_ref[pB + r, pl.ds(c * L, L)] = v
            return c_

        def lower_row(r, c_):
            for c in range(QC):
                v = plsc.load_gather(stage_ref, [pB + qvs[c], sh_v[c] + r])
                outbuf_ref[pB + r, pl.ds(c * L, L)] = v
            return c_

        def diag_row(r, c_):
            s = seg_start(mn + r, mx)
            sh = s - lax.shift_left(
                jnp.maximum(lax.shift_right_arithmetic(s, 7), 0), 7
            )
            rv = jnp.full((L,), pB + r, dtype=jnp.int32)
            for c in range(QC):
                xu = jnp.maximum(sh + qvs[c], 0)
                vu = plsc.load_gather(stage_ref, [rv, xu])
                xl = jnp.maximum(sh_v[c] + r, 0)
                vl = plsc.load_gather(stage_ref, [pB + qvs[c], xl])
                val = jnp.where(qvs[c] > r, vu, jnp.where(qvs[c] < r, vl, 1.0))
                outbuf_ref[pB + r, pl.ds(c * L, L)] = val
            return c_

        @pl.when(bi_lt)
        def _():
            lax.fori_loop(0, B, upper_row, 0)

        @pl.when(bi_gt)
        def _():
            lax.fori_loop(0, B, lower_row, 0)

        @pl.when(jnp.logical_not(jnp.logical_or(bi_lt, bi_gt)))
        def _():
            lax.fori_loop(0, B, diag_row, 0)

    build_and_fire(0, 0)
    build_and_fire(1, 1)

    def step(j, carry):
        for p in range(2):
            blk = 2 * j + p
            wait_stage(p)

            @pl.when(j > 0)
            def _():
                out_copy(blk - 2, p).wait()

            compute_block(blk, p)
            out_copy(blk, p).start()

            @pl.when(j < BLOCKS_PER_W // 2 - 1)
            def _():
                build_and_fire(blk + 2, p)
        return carry

    lax.fori_loop(0, BLOCKS_PER_W // 2, step, 0)
    out_copy(BLOCKS_PER_W - 2, 0).wait()
    out_copy(BLOCKS_PER_W - 1, 1).wait()


@jax.jit
def kernel(compressed_matrix):
    comp2 = compressed_matrix.reshape(ROWS, G)
    mesh = plsc.VectorSubcoreMesh(core_axis_name="c", subcore_axis_name="s")
    run = pl.kernel(
        _body,
        out_type=jax.ShapeDtypeStruct((N, N), jnp.float32),
        mesh=mesh,
        scratch_types=[
            pltpu.VMEM((2, NSEG, B), jnp.int32),
            pltpu.VMEM((2 * B, NSEG * G), jnp.float32),
            pltpu.VMEM((2 * B, B), jnp.float32),
            pltpu.VMEM((2, B), jnp.int32),
            pltpu.SemaphoreType.DMA,
            pltpu.SemaphoreType.DMA,
            pltpu.SemaphoreType.DMA,
            pltpu.SemaphoreType.DMA,
        ],
        compiler_params=pltpu.CompilerParams(needs_layout_passes=False),
    )
    return run(comp2)
